# Initial kernel scaffold; baseline (speedup 1.0000x reference)
import jax
import jax.numpy as jnp
from jax.experimental import pallas as pl


def kernel(node_features, incidence_matrix, W_ne, b_ne, W_en, b_en):
    raise NotImplementedError("write your pallas kernel here")


# speedup vs baseline: 850.8937x; 850.8937x over previous
import jax
import jax.numpy as jnp
from jax.experimental import pallas as pl


def _body(a_ref, x_ref, wne_ref, bne_ref, wen_ref, ben_ref, y_ref, h_ref):
    A = a_ref[:]
    X = x_ref[:]
    sums = jax.lax.dot_general(
        A, X, (((1,), (0,)), ((), ())), preferred_element_type=jnp.float32)
    counts = jnp.sum(A, axis=1, keepdims=True)
    mean = sums / jnp.maximum(counts, 1.0)
    H = jnp.dot(mean, wne_ref[:], preferred_element_type=jnp.float32) + bne_ref[:]
    G = jnp.dot(H, wen_ref[:], preferred_element_type=jnp.float32)
    Y = jax.lax.dot_general(
        A, G, (((0,), (0,)), ((), ())), preferred_element_type=jnp.float32)
    y_ref[:] = Y + ben_ref[:] + x_ref[:]
    h_ref[:] = H


def kernel(node_features, incidence_matrix, W_ne, b_ne, W_en, b_en):
    n_edges = incidence_matrix.shape[0]
    n_nodes, in_ch = node_features.shape
    out_ch = W_ne.shape[1]
    y, h = pl.pallas_call(
        _body,
        out_shape=(
            jax.ShapeDtypeStruct((n_nodes, out_ch), jnp.float32),
            jax.ShapeDtypeStruct((n_edges, out_ch), jnp.float32),
        ),
    )(incidence_matrix, node_features, W_ne, b_ne.reshape(1, -1),
      W_en, b_en.reshape(1, -1))
    attention_weights = jnp.ones((n_edges,), dtype=jnp.float32)
    return (y, h, attention_weights)
